# baseline (speedup 1.0000x reference)
import functools

import numpy as np
import jax
import jax.numpy as jnp
from jax import lax
from jax.experimental import pallas as pl
from jax.experimental.pallas import tpu as pltpu
from jax.experimental.pallas import tpu_sc as plsc

_B, _H, _W, _C = 32, 512, 512, 3
_NP = _B * _C
_CHUNK = 32
_NCHUNK = _H // _CHUNK
_NW = 32

_FRAC_LO, _FRAC_HI, _RATIO = 0.05, 0.1, 0.3


def _erase_rects(batch, height, width):
    rng = np.random.default_rng(0)
    area = float(height * width)
    rects = []
    for _ in range(batch):
        target_area = rng.uniform(_FRAC_LO, _FRAC_HI) * area
        target_ratio = rng.uniform(_RATIO, 1.0 / _RATIO)
        th = int(round(float(np.sqrt(target_area)) * target_ratio))
        tw = int(round(float(np.sqrt(target_area)) / target_ratio))
        if tw < width and th < height:
            x0 = int(rng.integers(0, width - tw))
            y0 = int(rng.integers(0, height - th))
            rects.append((y0, x0, th, tw))
        else:
            rects.append(None)
    return rects


def _build_tasks():
    rects = _erase_rects(_B, _H, _W)
    t_tasks, s_tasks = [], []
    for img in range(_B):
        r = rects[img]
        y0, x0 = (r[0], r[1]) if r else (0, 0)
        y1, x1 = (r[0] + r[2], r[1] + r[3]) if r else (0, 0)
        for p in range(_C):
            plane = img * _C + p
            for c in range(_NCHUNK):
                lo = c * _CHUNK
                r0 = min(max(y0 - lo, 0), _CHUNK)
                r1 = min(max(y1 - lo, 0), _CHUNK)
                if r1 > r0:
                    t_tasks.append((plane, lo, r0, r1, x0, x1))
                else:
                    s_tasks.append((plane, lo))
    nt_max = -(-len(t_tasks) // _NW)
    per_t = [[] for _ in range(_NW)]
    for i, t in enumerate(t_tasks):
        per_t[i % _NW].append(t)
    s_iter = iter(s_tasks)
    for w in range(_NW):
        while len(per_t[w]) < nt_max:
            plane, lo = next(s_iter)
            per_t[w].append((plane, lo, 0, 0, 0, 0))
    rest = list(s_iter)
    ns_max = len(rest) // _NW
    assert ns_max * _NW == len(rest)
    per_s = [[] for _ in range(_NW)]
    for i, t in enumerate(rest):
        per_s[i % _NW].append(t)
    return nt_max, ns_max, per_t, per_s


_NT, _NS, _PER_T, _PER_S = _build_tasks()
_ROWLEN = -(-(_NT * 6 + _NS * 2) // 128) * 128


def _build_table():
    tab = np.zeros((_NW, 1, _ROWLEN), dtype=np.int32)
    for w in range(_NW):
        flat = [v for t in _PER_T[w] for v in t] + \
               [v for t in _PER_S[w] for v in t]
        tab[w, 0, :len(flat)] = flat
    return tab


_TABLE_NP = _build_table()

_ORDER = []
_ti, _si = 0, 0
while _ti < _NT or _si < _NS:
    take_t = _ti < _NT and (_si >= _NS or _ti * _NS <= _si * _NT)
    if take_t:
        _ORDER.append(("T", _ti))
        _ti += 1
    else:
        _ORDER.append(("S", _si))
        _si += 1

_mesh = plsc.VectorSubcoreMesh(core_axis_name="c", subcore_axis_name="s")

_NBUF = 4
_SPB = 3
_LOOKAHEAD = 4


@functools.partial(
    pl.kernel,
    mesh=_mesh,
    out_type=jax.ShapeDtypeStruct((_NP, _H, _W), jnp.float32),
    scratch_types=(
        [pltpu.VMEM((_CHUNK, _W), jnp.float32) for _ in range(_NBUF)]
        + [pltpu.VMEM((1, _ROWLEN), jnp.int32)]
        + [pltpu.VMEM_SHARED((16, _SPB, _CHUNK, _W), jnp.float32)]
        + [pltpu.SemaphoreType.DMA for _ in range(2 * _NBUF + 2 * _SPB)]
    ),
    compiler_params=pltpu.CompilerParams(needs_layout_passes=False),
)
def _erase_sc(x_hbm, prm_hbm, out_hbm, *scratch):
    bufs = scratch[:_NBUF]
    prm = scratch[_NBUF]
    sp = scratch[_NBUF + 1]
    sems = scratch[_NBUF + 2:]
    isems = sems[:_NBUF]
    osems = sems[_NBUF:2 * _NBUF]
    sisems = sems[2 * _NBUF:2 * _NBUF + _SPB]
    sosems = sems[2 * _NBUF + _SPB:]
    wid = lax.axis_index("s") * 2 + lax.axis_index("c")
    sid = lax.axis_index("s")
    pltpu.sync_copy(prm_hbm.at[wid], prm)
    lane = lax.iota(jnp.int32, 16)

    def _scalar(pos):
        v = prm[0, pl.ds((pos // 16) * 16, 16)]
        return jnp.max(jnp.where(lane == (pos % 16), v, 0))

    t_prm = []
    for s in range(_NT):
        base = s * 6
        t_prm.append(tuple(_scalar(base + i) for i in range(6)))
    s_prm = []
    for s in range(_NS):
        base = _NT * 6 + s * 2
        s_prm.append(tuple(_scalar(base + i) for i in range(2)))

    def _src(path, idx):
        if path == "T":
            plane, lo = t_prm[idx][:2]
            return x_hbm.at[plane, pl.ds(pl.multiple_of(lo, 8), _CHUNK)]
        plane, lo = s_prm[idx]
        return x_hbm.at[plane, pl.ds(pl.multiple_of(lo, 8), _CHUNK)]

    def _dst(path, idx):
        if path == "T":
            plane, lo = t_prm[idx][:2]
            return out_hbm.at[plane, pl.ds(pl.multiple_of(lo, 8), _CHUNK)]
        plane, lo = s_prm[idx]
        return out_hbm.at[plane, pl.ds(pl.multiple_of(lo, 8), _CHUNK)]

    def _buf(path, idx):
        if path == "T":
            return bufs[idx % _NBUF]
        return sp.at[sid, idx % _SPB]

    def _isem(path, idx):
        return isems[idx % _NBUF] if path == "T" else sisems[idx % _SPB]

    def _osem(path, idx):
        return osems[idx % _NBUF] if path == "T" else sosems[idx % _SPB]

    def _zero(idx):
        buf = bufs[idx % _NBUF]
        _, _, r0, r1, x0, x1 = t_prm[idx]
        j0 = x0 >> 4
        j1 = (jnp.maximum(x1, 1) - 1) >> 4

        def _row(r, _):
            def _col(j, _):
                col = lane + (j << 4)
                m = (col >= x0) & (col < x1)
                v = buf[r, pl.ds(jnp.int32(j << 4), 16)]
                buf[r, pl.ds(jnp.int32(j << 4), 16)] = jnp.where(
                    m, jnp.float32(0), v)
                return 0

            lax.fori_loop(j0, j1 + 1, _col, 0, unroll=False)
            return 0

        lax.fori_loop(r0, r1, _row, 0, unroll=False)

    n = len(_ORDER)
    ring = {"T": _NBUF, "S": _SPB}
    prev_same_buf = {}
    last_seen = {}
    for k, (path, idx) in enumerate(_ORDER):
        key = (path, idx % ring[path])
        if key in last_seen:
            prev_same_buf[k] = last_seen[key]
        last_seen[key] = k

    h_in, h_out = {}, {}
    drained = set()
    state = {"next": 0}

    def _pump(done_out):
        while state["next"] < n and state["next"] <= done_out + _LOOKAHEAD:
            k = state["next"]
            pk = prev_same_buf.get(k)
            if pk is not None and pk > done_out:
                break
            if pk is not None:
                h_out[pk].wait()
                drained.add(pk)
            path, idx = _ORDER[k]
            h_in[k] = pltpu.async_copy(_src(path, idx), _buf(path, idx),
                                       _isem(path, idx))
            state["next"] += 1

    _pump(-1)
    for k in range(n):
        path, idx = _ORDER[k]
        h_in[k].wait()
        if path == "T":
            _zero(idx)
        h_out[k] = pltpu.async_copy(_buf(path, idx), _dst(path, idx),
                                    _osem(path, idx))
        _pump(k)
    for k in range(n):
        if k not in drained:
            h_out[k].wait()


def kernel(inputs):
    x = inputs.transpose(0, 3, 1, 2).reshape(_NP, _H, _W)
    out = _erase_sc(x, jnp.asarray(_TABLE_NP))
    return out.reshape(_B, _C, _H, _W).transpose(0, 2, 3, 1)


# speedup vs baseline: 1.0056x; 1.0056x over previous
import functools

import numpy as np
import jax
import jax.numpy as jnp
from jax import lax
from jax.experimental import pallas as pl
from jax.experimental.pallas import tpu as pltpu
from jax.experimental.pallas import tpu_sc as plsc

_B, _H, _W, _C = 32, 512, 512, 3
_NP = _B * _C
_CHUNK = 32
_NCHUNK = _H // _CHUNK
_NW = 32

_FRAC_LO, _FRAC_HI, _RATIO = 0.05, 0.1, 0.3


def _erase_rects(batch, height, width):
    rng = np.random.default_rng(0)
    area = float(height * width)
    rects = []
    for _ in range(batch):
        target_area = rng.uniform(_FRAC_LO, _FRAC_HI) * area
        target_ratio = rng.uniform(_RATIO, 1.0 / _RATIO)
        th = int(round(float(np.sqrt(target_area)) * target_ratio))
        tw = int(round(float(np.sqrt(target_area)) / target_ratio))
        if tw < width and th < height:
            x0 = int(rng.integers(0, width - tw))
            y0 = int(rng.integers(0, height - th))
            rects.append((y0, x0, th, tw))
        else:
            rects.append(None)
    return rects


def _build_tasks():
    rects = _erase_rects(_B, _H, _W)
    t_tasks, s_tasks = [], []
    for img in range(_B):
        r = rects[img]
        y0, x0 = (r[0], r[1]) if r else (0, 0)
        y1, x1 = (r[0] + r[2], r[1] + r[3]) if r else (0, 0)
        for p in range(_C):
            plane = img * _C + p
            for c in range(_NCHUNK):
                lo = c * _CHUNK
                r0 = min(max(y0 - lo, 0), _CHUNK)
                r1 = min(max(y1 - lo, 0), _CHUNK)
                if r1 > r0:
                    t_tasks.append((plane, lo, r0, r1, x0, x1))
                else:
                    s_tasks.append((plane, lo))
    nt_max = -(-len(t_tasks) // _NW)
    per_t = [[] for _ in range(_NW)]
    for i, t in enumerate(t_tasks):
        per_t[i % _NW].append(t)
    s_iter = iter(s_tasks)
    for w in range(_NW):
        while len(per_t[w]) < nt_max:
            plane, lo = next(s_iter)
            per_t[w].append((plane, lo, 0, 0, 0, 0))
    rest = list(s_iter)
    ns_max = len(rest) // _NW
    assert ns_max * _NW == len(rest)
    per_s = [[] for _ in range(_NW)]
    for i, t in enumerate(rest):
        per_s[i % _NW].append(t)
    return nt_max, ns_max, per_t, per_s


_NT, _NS, _PER_T, _PER_S = _build_tasks()
_ROWLEN = -(-(_NT * 6 + _NS * 2) // 128) * 128


def _build_table():
    tab = np.zeros((_NW, 1, _ROWLEN), dtype=np.int32)
    for w in range(_NW):
        flat = [v for t in _PER_T[w] for v in t] + \
               [v for t in _PER_S[w] for v in t]
        tab[w, 0, :len(flat)] = flat
    return tab


_TABLE_NP = _build_table()

_ORDER = []
_ti, _si = 0, 0
while _ti < _NT or _si < _NS:
    take_t = _ti < _NT and (_si >= _NS or _ti * _NS <= _si * _NT)
    if take_t:
        _ORDER.append(("T", _ti))
        _ti += 1
    else:
        _ORDER.append(("S", _si))
        _si += 1

_mesh = plsc.VectorSubcoreMesh(core_axis_name="c", subcore_axis_name="s")

_NBUF = 5
_SPB = 2
_LOOKAHEAD = 4


@functools.partial(
    pl.kernel,
    mesh=_mesh,
    out_type=jax.ShapeDtypeStruct((_NP, _H, _W), jnp.float32),
    scratch_types=(
        [pltpu.VMEM((_CHUNK, _W), jnp.float32) for _ in range(_NBUF)]
        + [pltpu.VMEM((1, _ROWLEN), jnp.int32)]
        + [pltpu.VMEM_SHARED((16, _SPB, _CHUNK, _W), jnp.float32)]
        + [pltpu.SemaphoreType.DMA for _ in range(2 * _NBUF + 2 * _SPB)]
    ),
    compiler_params=pltpu.CompilerParams(needs_layout_passes=False),
)
def _erase_sc(x_hbm, prm_hbm, out_hbm, *scratch):
    bufs = scratch[:_NBUF]
    prm = scratch[_NBUF]
    sp = scratch[_NBUF + 1]
    sems = scratch[_NBUF + 2:]
    isems = sems[:_NBUF]
    osems = sems[_NBUF:2 * _NBUF]
    sisems = sems[2 * _NBUF:2 * _NBUF + _SPB]
    sosems = sems[2 * _NBUF + _SPB:]
    wid = lax.axis_index("s") * 2 + lax.axis_index("c")
    sid = lax.axis_index("s")
    pltpu.sync_copy(prm_hbm.at[wid], prm)
    lane = lax.iota(jnp.int32, 16)

    def _scalar(pos):
        v = prm[0, pl.ds((pos // 16) * 16, 16)]
        return jnp.max(jnp.where(lane == (pos % 16), v, 0))

    t_prm = []
    for s in range(_NT):
        base = s * 6
        t_prm.append(tuple(_scalar(base + i) for i in range(6)))
    s_prm = []
    for s in range(_NS):
        base = _NT * 6 + s * 2
        s_prm.append(tuple(_scalar(base + i) for i in range(2)))

    def _src(path, idx):
        if path == "T":
            plane, lo = t_prm[idx][:2]
            return x_hbm.at[plane, pl.ds(pl.multiple_of(lo, 8), _CHUNK)]
        plane, lo = s_prm[idx]
        return x_hbm.at[plane, pl.ds(pl.multiple_of(lo, 8), _CHUNK)]

    def _dst(path, idx):
        if path == "T":
            plane, lo = t_prm[idx][:2]
            return out_hbm.at[plane, pl.ds(pl.multiple_of(lo, 8), _CHUNK)]
        plane, lo = s_prm[idx]
        return out_hbm.at[plane, pl.ds(pl.multiple_of(lo, 8), _CHUNK)]

    def _buf(path, idx):
        if path == "T":
            return bufs[idx % _NBUF]
        return sp.at[sid, idx % _SPB]

    def _isem(path, idx):
        return isems[idx % _NBUF] if path == "T" else sisems[idx % _SPB]

    def _osem(path, idx):
        return osems[idx % _NBUF] if path == "T" else sosems[idx % _SPB]

    def _zero(idx):
        buf = bufs[idx % _NBUF]
        _, _, r0, r1, x0, x1 = t_prm[idx]
        j0 = x0 >> 4
        j1 = (jnp.maximum(x1, 1) - 1) >> 4

        def _row(r, _):
            def _col(j, _):
                col = lane + (j << 4)
                m = (col >= x0) & (col < x1)
                v = buf[r, pl.ds(jnp.int32(j << 4), 16)]
                buf[r, pl.ds(jnp.int32(j << 4), 16)] = jnp.where(
                    m, jnp.float32(0), v)
                return 0

            lax.fori_loop(j0, j1 + 1, _col, 0, unroll=False)
            return 0

        lax.fori_loop(r0, r1, _row, 0, unroll=False)

    n = len(_ORDER)
    ring = {"T": _NBUF, "S": _SPB}
    prev_same_buf = {}
    last_seen = {}
    for k, (path, idx) in enumerate(_ORDER):
        key = (path, idx % ring[path])
        if key in last_seen:
            prev_same_buf[k] = last_seen[key]
        last_seen[key] = k

    h_in, h_out = {}, {}
    drained = set()
    state = {"next": 0}

    def _pump(done_out):
        while state["next"] < n and state["next"] <= done_out + _LOOKAHEAD:
            k = state["next"]
            pk = prev_same_buf.get(k)
            if pk is not None and pk > done_out:
                break
            if pk is not None:
                h_out[pk].wait()
                drained.add(pk)
            path, idx = _ORDER[k]
            h_in[k] = pltpu.async_copy(_src(path, idx), _buf(path, idx),
                                       _isem(path, idx))
            state["next"] += 1

    _pump(-1)
    for k in range(n):
        path, idx = _ORDER[k]
        h_in[k].wait()
        if path == "T":
            _zero(idx)
        h_out[k] = pltpu.async_copy(_buf(path, idx), _dst(path, idx),
                                    _osem(path, idx))
        _pump(k)
    for k in range(n):
        if k not in drained:
            h_out[k].wait()


def kernel(inputs):
    x = inputs.transpose(0, 3, 1, 2).reshape(_NP, _H, _W)
    out = _erase_sc(x, jnp.asarray(_TABLE_NP))
    return out.reshape(_B, _C, _H, _W).transpose(0, 2, 3, 1)
